# baseline (speedup 1.0000x reference)
import functools
import math

import jax
import jax.numpy as jnp
from jax import lax
from jax.experimental import pallas as pl
from jax.experimental.pallas import tpu as pltpu
from jax.experimental.pallas import tpu_sc as plsc

B, T = 2, 2048
N = B * T
D = 1024
E = 8
K = 2
H = 1408
CAP = int(math.ceil(1.25 * K * N / E))
AUXC = 0.01 * 8.0

TRASH = E * CAP
EIN_ROWS = E * CAP + 8

NC, NS, L = 2, 16, 16
NW = NC * NS
TOK_W = N // NW
CH = 32
NCHUNK = TOK_W // CH

ROUTE_BLK = 512
ROUTE_GRID = N // ROUTE_BLK


def _route_body(x_ref, rw_ref, cmbi_ref, cmbf_ref, aux_ref,
                carry, psum, cnt):
    i = pl.program_id(0)

    @pl.when(i == 0)
    def _():
        carry[...] = jnp.zeros_like(carry)
        psum[...] = jnp.zeros_like(psum)
        cnt[...] = jnp.zeros_like(cnt)

    xb = x_ref[...]
    logits = lax.dot_general(xb, rw_ref[...], (((1,), (1,)), ((), ())),
                             preferred_element_type=jnp.float32)
    m = jnp.max(logits, axis=1, keepdims=True)
    ex = jnp.exp(logits - m)
    probs = ex / jnp.sum(ex, axis=1, keepdims=True)

    lane = lax.broadcasted_iota(jnp.int32, (ROUTE_BLK, E), 1)
    p0 = jnp.max(probs, axis=1, keepdims=True)
    e0 = jnp.min(jnp.where(probs == p0, lane, E), axis=1, keepdims=True)
    oh0 = (lane == e0).astype(jnp.float32)
    probs1 = jnp.where(lane == e0, -1.0, probs)
    p1 = jnp.max(probs1, axis=1, keepdims=True)
    e1 = jnp.min(jnp.where(probs1 == p1, lane, E), axis=1, keepdims=True)
    oh1 = (lane == e1).astype(jnp.float32)

    denom = p0 + p1 + 1e-9
    w0 = p0 / denom
    w1 = p1 / denom

    S = oh0 + oh1
    r = lax.broadcasted_iota(jnp.int32, (ROUTE_BLK, ROUTE_BLK), 0)
    c = lax.broadcasted_iota(jnp.int32, (ROUTE_BLK, ROUTE_BLK), 1)
    tri = (c < r).astype(jnp.float32)
    ec = jnp.dot(tri, S, preferred_element_type=jnp.float32) + carry[...]
    pos0 = jnp.sum(ec * oh0, axis=1, keepdims=True)
    pos1 = jnp.sum(ec * oh1, axis=1, keepdims=True)
    keep0 = (pos0 < CAP).astype(jnp.float32)
    keep1 = (pos1 < CAP).astype(jnp.float32)
    slot0 = e0 * CAP + jnp.minimum(pos0, CAP - 1).astype(jnp.int32)
    slot1 = e1 * CAP + jnp.minimum(pos1, CAP - 1).astype(jnp.int32)

    w0k = w0 * keep0
    w1k = w1 * keep1
    sc0 = jnp.where(pos0 < CAP, slot0, TRASH)
    sc1 = jnp.where(pos1 < CAP, slot1, TRASH)
    cmbi = jnp.concatenate(
        [
            jnp.broadcast_to(sc0, (ROUTE_BLK, L)),
            jnp.broadcast_to(sc1, (ROUTE_BLK, L)),
            jnp.broadcast_to(slot0, (ROUTE_BLK, L)),
            jnp.broadcast_to(slot1, (ROUTE_BLK, L)),
            jnp.zeros((ROUTE_BLK, 4 * L), jnp.int32),
        ],
        axis=1,
    )
    cmbi_ref[...] = cmbi.reshape(1, ROUTE_BLK, 8 * L)
    cmbf = jnp.concatenate(
        [
            jnp.broadcast_to(w0k, (ROUTE_BLK, L)),
            jnp.broadcast_to(w1k, (ROUTE_BLK, L)),
            jnp.zeros((ROUTE_BLK, 6 * L), jnp.float32),
        ],
        axis=1,
    )
    cmbf_ref[...] = cmbf.reshape(1, ROUTE_BLK, 8 * L)

    carry[...] = carry[...] + jnp.sum(S, axis=0, keepdims=True)
    psum[...] = psum[...] + jnp.sum(probs, axis=0, keepdims=True)
    cnt[...] = cnt[...] + jnp.sum(oh0, axis=0, keepdims=True)

    @pl.when(i == ROUTE_GRID - 1)
    def _():
        f = cnt[...] / jnp.float32(N)
        p = psum[...] / jnp.float32(N)
        aux_ref[...] = (jnp.sum(f * p) * jnp.float32(AUXC)).reshape(1, 1)


def _route(x2d, rw):
    return pl.pallas_call(
        _route_body,
        grid=(ROUTE_GRID,),
        in_specs=[
            pl.BlockSpec((ROUTE_BLK, D), lambda i: (i, 0)),
            pl.BlockSpec((E, D), lambda i: (0, 0)),
        ],
        out_specs=[
            pl.BlockSpec((1, ROUTE_BLK, 8 * L), lambda i: (i, 0, 0)),
            pl.BlockSpec((1, ROUTE_BLK, 8 * L), lambda i: (i, 0, 0)),
            pl.BlockSpec((1, 1), lambda i: (0, 0)),
        ],
        out_shape=[
            jax.ShapeDtypeStruct((ROUTE_GRID, ROUTE_BLK, 8 * L), jnp.int32),
            jax.ShapeDtypeStruct((ROUTE_GRID, ROUTE_BLK, 8 * L), jnp.float32),
            jax.ShapeDtypeStruct((1, 1), jnp.float32),
        ],
        scratch_shapes=[
            pltpu.VMEM((1, E), jnp.float32),
            pltpu.VMEM((1, E), jnp.float32),
            pltpu.VMEM((1, E), jnp.float32),
        ],
    )(x2d, rw)


@functools.lru_cache(maxsize=None)
def _sc_mesh():
    return plsc.VectorSubcoreMesh(core_axis_name="c", subcore_axis_name="s",
                                  num_cores=NC, num_subcores=NS)


CHD = 16
NCHD = TOK_W // CHD


def _lanes16():
    return lax.broadcasted_iota(jnp.int32, (L,), 0)


def _dcol(ref, c):
    lanes = _lanes16()
    acc = jnp.zeros((L,), jnp.int32)
    for r in range(L):
        acc = jnp.where(lanes == r, ref[r, pl.ds(c, L)], acc)
    return acc


def _dispatch_body(x_hbm, cmb_hbm, ein_hbm,
                   xb0, xb1, ma, mb, c0a, c0b, c1a, c1b, ls0, ls1, ss0, ss1):
    wid = lax.axis_index("s") * NC + lax.axis_index("c")
    XB = (xb0, xb1)
    M = (ma, mb)
    C0 = (c0a, c0b)
    C1 = (c1a, c1b)
    LS = (ls0, ls1)
    SS = (ss0, ss1)

    def issue_loads(ci):
        b = ci & 1
        base = wid * TOK_W + ci * CHD
        return [
            pltpu.async_copy(x_hbm.at[pl.ds(base, CHD)], XB[b], LS[b]),
            pltpu.async_copy(cmb_hbm.at[pl.ds(base, CHD)], M[b], LS[b]),
        ]

    loads = issue_loads(0)
    scatters = []
    for ci in range(NCHD):
        b = ci & 1
        for cp in loads:
            cp.wait()
        C0[b][...] = _dcol(M[b], 0)
        C1[b][...] = _dcol(M[b], L)
        new_scatters = [
            pltpu.async_copy(XB[b], ein_hbm.at[C0[b]], SS[b]),
            pltpu.async_copy(XB[b], ein_hbm.at[C1[b]], SS[b]),
        ]
        for cp in scatters:
            cp.wait()
        scatters = new_scatters
        if ci + 1 < NCHD:
            loads = issue_loads(ci + 1)
    for cp in scatters:
        cp.wait()


@functools.lru_cache(maxsize=None)
def _dispatch_kernel():
    return pl.kernel(
        _dispatch_body,
        out_type=jax.ShapeDtypeStruct((EIN_ROWS, D), jnp.float32),
        mesh=_sc_mesh(),
        scratch_types=(
            [pltpu.VMEM((CHD, D), jnp.float32)] * 2
            + [pltpu.VMEM((CHD, 8 * L), jnp.int32)] * 2
            + [pltpu.VMEM((CHD,), jnp.int32)] * 4
            + [pltpu.SemaphoreType.DMA] * 4
        ),
    )


FFN_BLK = 1280


def _ffn_body(xin_ref, wfc_ref, wpj_ref, out_ref):
    h = jnp.dot(xin_ref[...], wfc_ref[0], preferred_element_type=jnp.float32)
    a = jnp.square(jnp.maximum(h, 0.0))
    out_ref[...] = jnp.dot(a, wpj_ref[0], preferred_element_type=jnp.float32)


def _routed_ffn(ein, w_fc, w_proj):
    rblk = CAP // FFN_BLK
    return pl.pallas_call(
        _ffn_body,
        grid=(E, rblk),
        in_specs=[
            pl.BlockSpec((FFN_BLK, D),
                         lambda e, r: (e * (CAP // FFN_BLK) + r, 0)),
            pl.BlockSpec((1, D, H), lambda e, r: (e, 0, 0)),
            pl.BlockSpec((1, H, D), lambda e, r: (e, 0, 0)),
        ],
        out_specs=pl.BlockSpec((FFN_BLK, D),
                               lambda e, r: (e * (CAP // FFN_BLK) + r, 0)),
        out_shape=jax.ShapeDtypeStruct((E * CAP, D), jnp.float32),
    )(ein, w_fc, w_proj)


SFFN_BLK = 1024


def _shared_ffn(x2d, ws_fc, ws_proj):
    return pl.pallas_call(
        _ffn_body,
        grid=(N // SFFN_BLK,),
        in_specs=[
            pl.BlockSpec((SFFN_BLK, D), lambda i: (i, 0)),
            pl.BlockSpec((1, D, H), lambda i: (0, 0, 0)),
            pl.BlockSpec((1, H, D), lambda i: (0, 0, 0)),
        ],
        out_specs=pl.BlockSpec((SFFN_BLK, D), lambda i: (i, 0)),
        out_shape=jax.ShapeDtypeStruct((N, D), jnp.float32),
    )(x2d, ws_fc, ws_proj)


def _combine_body(eo_hbm, sh_hbm, cmbi_hbm, cmbf_hbm, y_hbm,
                  r0a, r0b, sha, shb, ybuf,
                  ma, mb, fa, fb, c0a, c0b,
                  ls0, ls1, gs0, gs1, ssem):
    wid = lax.axis_index("s") * NC + lax.axis_index("c")
    R01 = (r0a, r0b)
    SH = (sha, shb)
    M = (ma, mb)
    F = (fa, fb)
    C01 = (c0a, c0b)
    LS = (ls0, ls1)
    GS = (gs0, gs1)

    def issue_loads(ci):
        b = ci & 1
        base = wid * TOK_W + ci * CHD
        return [
            pltpu.async_copy(cmbi_hbm.at[pl.ds(base, CHD)], M[b], LS[b]),
            pltpu.async_copy(cmbf_hbm.at[pl.ds(base, CHD)], F[b], LS[b]),
            pltpu.async_copy(sh_hbm.at[pl.ds(base, CHD)], SH[b], LS[b]),
        ]

    def compute_store(ci, store_prev):
        b = ci & 1

        def row_fn(row, carry):
            w0v = F[b][row, pl.ds(0, L)]
            w1v = F[b][row, pl.ds(L, L)]
            for d in range(0, D, L):
                sl = pl.ds(d, L)
                ybuf[row, sl] = (SH[b][row, sl] + w0v * R01[b][row, sl]
                                 + w1v * R01[b][row + CHD, sl])
            return carry

        if store_prev is not None:
            store_prev.wait()
        lax.fori_loop(0, CHD, row_fn, 0)
        base = wid * TOK_W + ci * CHD
        return pltpu.async_copy(ybuf, y_hbm.at[pl.ds(base, CHD)], ssem)

    loads = issue_loads(0)
    gathers = []
    store = None
    for ci in range(NCHD):
        b = ci & 1
        for cp in loads:
            cp.wait()
        C01[b][pl.ds(0, CHD)] = _dcol(M[b], 2 * L)
        C01[b][pl.ds(CHD, CHD)] = _dcol(M[b], 3 * L)
        new_gathers = [
            pltpu.async_copy(eo_hbm.at[C01[b]], R01[b], GS[b]),
        ]
        if ci >= 1:
            for cp in gathers:
                cp.wait()
            store = compute_store(ci - 1, store)
        gathers = new_gathers
        if ci + 1 < NCHD:
            loads = issue_loads(ci + 1)
    for cp in gathers:
        cp.wait()
    store = compute_store(NCHD - 1, store)
    store.wait()


@functools.lru_cache(maxsize=None)
def _combine_kernel():
    return pl.kernel(
        _combine_body,
        out_type=jax.ShapeDtypeStruct((N, D), jnp.float32),
        mesh=_sc_mesh(),
        scratch_types=(
            [pltpu.VMEM((2 * CHD, D), jnp.float32)] * 2
            + [pltpu.VMEM((CHD, D), jnp.float32)] * 3
            + [pltpu.VMEM((CHD, 8 * L), jnp.int32)] * 2
            + [pltpu.VMEM((CHD, 8 * L), jnp.float32)] * 2
            + [pltpu.VMEM((2 * CHD,), jnp.int32)] * 2
            + [pltpu.SemaphoreType.DMA] * 5
        ),
    )


def kernel(x, router_weight, w_fc, w_proj, ws_fc, ws_proj):
    x2d = x.reshape(N, D)

    cmbi_o, cmbf_o, aux_o = _route(x2d, router_weight)
    cmbi = cmbi_o.reshape(N, 8 * L)
    cmbf = cmbf_o.reshape(N, 8 * L)

    ein = _dispatch_kernel()(x2d, cmbi)
    eo = _routed_ffn(ein, w_fc, w_proj)
    sh = _shared_ffn(x2d, ws_fc, ws_proj)
    y = _combine_kernel()(eo, sh, cmbi, cmbf)

    return y.reshape(B, T, D), aux_o.reshape(())


# speedup vs baseline: 1.0248x; 1.0248x over previous
import functools
import math

import jax
import jax.numpy as jnp
from jax import lax
from jax.experimental import pallas as pl
from jax.experimental.pallas import tpu as pltpu
from jax.experimental.pallas import tpu_sc as plsc

B, T = 2, 2048
N = B * T
D = 1024
E = 8
K = 2
H = 1408
CAP = int(math.ceil(1.25 * K * N / E))
AUXC = 0.01 * 8.0

TRASH = E * CAP
EIN_ROWS = E * CAP + 8

NC, NS, L = 2, 16, 16
NW = NC * NS
TOK_W = N // NW
CH = 32
NCHUNK = TOK_W // CH

ROUTE_BLK = 512
ROUTE_GRID = N // ROUTE_BLK


def _route_body(x_ref, rw_ref, cmbi_ref, cmbf_ref, aux_ref,
                carry, psum, cnt):
    i = pl.program_id(0)

    @pl.when(i == 0)
    def _():
        carry[...] = jnp.zeros_like(carry)
        psum[...] = jnp.zeros_like(psum)
        cnt[...] = jnp.zeros_like(cnt)

    xb = x_ref[...]
    logits = lax.dot_general(xb, rw_ref[...], (((1,), (1,)), ((), ())),
                             preferred_element_type=jnp.float32)
    m = jnp.max(logits, axis=1, keepdims=True)
    ex = jnp.exp(logits - m)
    probs = ex / jnp.sum(ex, axis=1, keepdims=True)

    lane = lax.broadcasted_iota(jnp.int32, (ROUTE_BLK, E), 1)
    p0 = jnp.max(probs, axis=1, keepdims=True)
    e0 = jnp.min(jnp.where(probs == p0, lane, E), axis=1, keepdims=True)
    oh0 = (lane == e0).astype(jnp.float32)
    probs1 = jnp.where(lane == e0, -1.0, probs)
    p1 = jnp.max(probs1, axis=1, keepdims=True)
    e1 = jnp.min(jnp.where(probs1 == p1, lane, E), axis=1, keepdims=True)
    oh1 = (lane == e1).astype(jnp.float32)

    denom = p0 + p1 + 1e-9
    w0 = p0 / denom
    w1 = p1 / denom

    S = oh0 + oh1
    r = lax.broadcasted_iota(jnp.int32, (ROUTE_BLK, ROUTE_BLK), 0)
    c = lax.broadcasted_iota(jnp.int32, (ROUTE_BLK, ROUTE_BLK), 1)
    tri = (c < r).astype(jnp.float32)
    ec = jnp.dot(tri, S, preferred_element_type=jnp.float32) + carry[...]
    pos0 = jnp.sum(ec * oh0, axis=1, keepdims=True)
    pos1 = jnp.sum(ec * oh1, axis=1, keepdims=True)
    keep0 = (pos0 < CAP).astype(jnp.float32)
    keep1 = (pos1 < CAP).astype(jnp.float32)
    slot0 = e0 * CAP + jnp.minimum(pos0, CAP - 1).astype(jnp.int32)
    slot1 = e1 * CAP + jnp.minimum(pos1, CAP - 1).astype(jnp.int32)

    w0k = w0 * keep0
    w1k = w1 * keep1
    sc0 = jnp.where(pos0 < CAP, slot0, TRASH)
    sc1 = jnp.where(pos1 < CAP, slot1, TRASH)
    cmbi = jnp.concatenate(
        [
            jnp.broadcast_to(sc0, (ROUTE_BLK, L)),
            jnp.broadcast_to(sc1, (ROUTE_BLK, L)),
            jnp.broadcast_to(slot0, (ROUTE_BLK, L)),
            jnp.broadcast_to(slot1, (ROUTE_BLK, L)),
            jnp.zeros((ROUTE_BLK, 4 * L), jnp.int32),
        ],
        axis=1,
    )
    cmbi_ref[...] = cmbi.reshape(1, ROUTE_BLK, 8 * L)
    cmbf = jnp.concatenate(
        [
            jnp.broadcast_to(w0k, (ROUTE_BLK, L)),
            jnp.broadcast_to(w1k, (ROUTE_BLK, L)),
            jnp.zeros((ROUTE_BLK, 6 * L), jnp.float32),
        ],
        axis=1,
    )
    cmbf_ref[...] = cmbf.reshape(1, ROUTE_BLK, 8 * L)

    carry[...] = carry[...] + jnp.sum(S, axis=0, keepdims=True)
    psum[...] = psum[...] + jnp.sum(probs, axis=0, keepdims=True)
    cnt[...] = cnt[...] + jnp.sum(oh0, axis=0, keepdims=True)

    @pl.when(i == ROUTE_GRID - 1)
    def _():
        f = cnt[...] / jnp.float32(N)
        p = psum[...] / jnp.float32(N)
        aux_ref[...] = (jnp.sum(f * p) * jnp.float32(AUXC)).reshape(1, 1)


def _route(x2d, rw):
    return pl.pallas_call(
        _route_body,
        grid=(ROUTE_GRID,),
        in_specs=[
            pl.BlockSpec((ROUTE_BLK, D), lambda i: (i, 0)),
            pl.BlockSpec((E, D), lambda i: (0, 0)),
        ],
        out_specs=[
            pl.BlockSpec((1, ROUTE_BLK, 8 * L), lambda i: (i, 0, 0)),
            pl.BlockSpec((1, ROUTE_BLK, 8 * L), lambda i: (i, 0, 0)),
            pl.BlockSpec((1, 1), lambda i: (0, 0)),
        ],
        out_shape=[
            jax.ShapeDtypeStruct((ROUTE_GRID, ROUTE_BLK, 8 * L), jnp.int32),
            jax.ShapeDtypeStruct((ROUTE_GRID, ROUTE_BLK, 8 * L), jnp.float32),
            jax.ShapeDtypeStruct((1, 1), jnp.float32),
        ],
        scratch_shapes=[
            pltpu.VMEM((1, E), jnp.float32),
            pltpu.VMEM((1, E), jnp.float32),
            pltpu.VMEM((1, E), jnp.float32),
        ],
    )(x2d, rw)


@functools.lru_cache(maxsize=None)
def _sc_mesh():
    return plsc.VectorSubcoreMesh(core_axis_name="c", subcore_axis_name="s",
                                  num_cores=NC, num_subcores=NS)


CHD = 16
NCHD = TOK_W // CHD


def _lanes16():
    return lax.broadcasted_iota(jnp.int32, (L,), 0)


def _dcol(ref, c):
    lanes = _lanes16()
    acc = jnp.zeros((L,), jnp.int32)
    for r in range(L):
        acc = jnp.where(lanes == r, ref[r, pl.ds(c, L)], acc)
    return acc


def _dispatch_body(x_hbm, cmb_hbm, ein_hbm,
                   xb0, xb1, ma, mb, c0a, c0b, c1a, c1b, ls0, ls1, ss0, ss1):
    wid = lax.axis_index("s") * NC + lax.axis_index("c")
    XB = (xb0, xb1)
    M = (ma, mb)
    C0 = (c0a, c0b)
    C1 = (c1a, c1b)
    LS = (ls0, ls1)
    SS = (ss0, ss1)

    def issue_loads(ci):
        b = ci & 1
        base = wid * TOK_W + ci * CHD
        return [
            pltpu.async_copy(x_hbm.at[pl.ds(base, CHD)], XB[b], LS[b]),
            pltpu.async_copy(cmb_hbm.at[pl.ds(base, CHD)], M[b], LS[b]),
        ]

    loads = issue_loads(0)
    scatters = []
    for ci in range(NCHD):
        b = ci & 1
        for cp in loads:
            cp.wait()
        C0[b][...] = _dcol(M[b], 0)
        C1[b][...] = _dcol(M[b], L)
        new_scatters = [
            pltpu.async_copy(XB[b], ein_hbm.at[C0[b]], SS[b]),
            pltpu.async_copy(XB[b], ein_hbm.at[C1[b]], SS[b]),
        ]
        for cp in scatters:
            cp.wait()
        scatters = new_scatters
        if ci + 1 < NCHD:
            loads = issue_loads(ci + 1)
    for cp in scatters:
        cp.wait()


@functools.lru_cache(maxsize=None)
def _dispatch_kernel():
    return pl.kernel(
        _dispatch_body,
        out_type=jax.ShapeDtypeStruct((EIN_ROWS, D), jnp.float32),
        mesh=_sc_mesh(),
        scratch_types=(
            [pltpu.VMEM((CHD, D), jnp.float32)] * 2
            + [pltpu.VMEM((CHD, 8 * L), jnp.int32)] * 2
            + [pltpu.VMEM((CHD,), jnp.int32)] * 4
            + [pltpu.SemaphoreType.DMA] * 4
        ),
    )


FFN_BLK = 1280


def _ffn_body(xin_ref, wfc_ref, wpj_ref, out_ref):
    h = jnp.dot(xin_ref[...], wfc_ref[0], preferred_element_type=jnp.float32)
    a = jnp.square(jnp.maximum(h, 0.0))
    out_ref[...] = jnp.dot(a, wpj_ref[0], preferred_element_type=jnp.float32)


def _routed_ffn(ein, w_fc, w_proj):
    rblk = CAP // FFN_BLK
    return pl.pallas_call(
        _ffn_body,
        grid=(E, rblk),
        in_specs=[
            pl.BlockSpec((FFN_BLK, D),
                         lambda e, r: (e * (CAP // FFN_BLK) + r, 0)),
            pl.BlockSpec((1, D, H), lambda e, r: (e, 0, 0)),
            pl.BlockSpec((1, H, D), lambda e, r: (e, 0, 0)),
        ],
        out_specs=pl.BlockSpec((FFN_BLK, D),
                               lambda e, r: (e * (CAP // FFN_BLK) + r, 0)),
        out_shape=jax.ShapeDtypeStruct((E * CAP, D), jnp.float32),
    )(ein, w_fc, w_proj)


SFFN_BLK = 1024


def _shared_ffn(x2d, ws_fc, ws_proj):
    return pl.pallas_call(
        _ffn_body,
        grid=(N // SFFN_BLK,),
        in_specs=[
            pl.BlockSpec((SFFN_BLK, D), lambda i: (i, 0)),
            pl.BlockSpec((1, D, H), lambda i: (0, 0, 0)),
            pl.BlockSpec((1, H, D), lambda i: (0, 0, 0)),
        ],
        out_specs=pl.BlockSpec((SFFN_BLK, D), lambda i: (i, 0)),
        out_shape=jax.ShapeDtypeStruct((N, D), jnp.float32),
    )(x2d, ws_fc, ws_proj)


def _combine_body(eo_hbm, sh_hbm, cmbi_hbm, cmbf_hbm, y_hbm,
                  r0a, r0b, r1a, r1b, sha, shb, ybuf,
                  ma, mb, fa, fb, c0a, c0b, c1a, c1b,
                  ls0, ls1, gs0, gs1, ssem):
    wid = lax.axis_index("s") * NC + lax.axis_index("c")
    R0 = (r0a, r0b)
    R1 = (r1a, r1b)
    SH = (sha, shb)
    M = (ma, mb)
    F = (fa, fb)
    C0 = (c0a, c0b)
    C1 = (c1a, c1b)
    LS = (ls0, ls1)
    GS = (gs0, gs1)

    def issue_loads(ci):
        b = ci & 1
        base = wid * TOK_W + ci * CHD
        return [
            pltpu.async_copy(cmbi_hbm.at[pl.ds(base, CHD)], M[b], LS[b]),
            pltpu.async_copy(cmbf_hbm.at[pl.ds(base, CHD)], F[b], LS[b]),
            pltpu.async_copy(sh_hbm.at[pl.ds(base, CHD)], SH[b], LS[b]),
        ]

    def compute_store(ci, store_prev):
        b = ci & 1

        def row_fn(row, carry):
            w0v = F[b][row, pl.ds(0, L)]
            w1v = F[b][row, pl.ds(L, L)]
            for d in range(0, D, L):
                sl = pl.ds(d, L)
                ybuf[row, sl] = (SH[b][row, sl] + w0v * R0[b][row, sl]
                                 + w1v * R1[b][row, sl])
            return carry

        if store_prev is not None:
            store_prev.wait()
        lax.fori_loop(0, CHD, row_fn, 0)
        base = wid * TOK_W + ci * CHD
        return pltpu.async_copy(ybuf, y_hbm.at[pl.ds(base, CHD)], ssem)

    loads = issue_loads(0)
    gathers = []
    store = None
    for ci in range(NCHD):
        b = ci & 1
        for cp in loads:
            cp.wait()
        C0[b][...] = _dcol(M[b], 2 * L)
        C1[b][...] = _dcol(M[b], 3 * L)
        new_gathers = [
            pltpu.async_copy(eo_hbm.at[C0[b]], R0[b], GS[b]),
            pltpu.async_copy(eo_hbm.at[C1[b]], R1[b], GS[b]),
        ]
        if ci >= 1:
            for cp in gathers:
                cp.wait()
            store = compute_store(ci - 1, store)
        gathers = new_gathers
        if ci + 1 < NCHD:
            loads = issue_loads(ci + 1)
    for cp in gathers:
        cp.wait()
    store = compute_store(NCHD - 1, store)
    store.wait()


@functools.lru_cache(maxsize=None)
def _combine_kernel():
    return pl.kernel(
        _combine_body,
        out_type=jax.ShapeDtypeStruct((N, D), jnp.float32),
        mesh=_sc_mesh(),
        scratch_types=(
            [pltpu.VMEM((CHD, D), jnp.float32)] * 7
            + [pltpu.VMEM((CHD, 8 * L), jnp.int32)] * 2
            + [pltpu.VMEM((CHD, 8 * L), jnp.float32)] * 2
            + [pltpu.VMEM((CHD,), jnp.int32)] * 4
            + [pltpu.SemaphoreType.DMA] * 5
        ),
    )


def kernel(x, router_weight, w_fc, w_proj, ws_fc, ws_proj):
    x2d = x.reshape(N, D)

    cmbi_o, cmbf_o, aux_o = _route(x2d, router_weight)
    cmbi = cmbi_o.reshape(N, 8 * L)
    cmbf = cmbf_o.reshape(N, 8 * L)

    ein = _dispatch_kernel()(x2d, cmbi)
    eo = _routed_ffn(ein, w_fc, w_proj)
    sh = _shared_ffn(x2d, ws_fc, ws_proj)
    y = _combine_kernel()(eo, sh, cmbi, cmbf)

    return y.reshape(B, T, D), aux_o.reshape(())
